# baseline (speedup 1.0000x reference)
import jax
import jax.numpy as jnp
from jax import lax
from jax.experimental import pallas as pl
from jax.experimental.pallas import tpu as pltpu

SIGMA = 5.0
B, C, H, W = 128, 3, 384, 384
INV = 1.0 / (2.0 * SIGMA * SIGMA)


def _body(mask_ref, gt_ref, img_ref, out_ref, heat_ref, sem):
    def step(b, c):
        @pl.when(mask_ref[b] != 0)
        def _():
            x0 = gt_ref[b, 0]
            y0 = gt_ref[b, 1]
            xs = lax.broadcasted_iota(jnp.int32, (1, 1, 1, W), 3).astype(jnp.float32)
            ys = lax.broadcasted_iota(jnp.int32, (1, 1, H, 1), 2).astype(jnp.float32)
            gx = jnp.exp(-((xs - x0) ** 2) * INV)
            gy = jnp.exp(-((ys - y0) ** 2) * INV)
            heat_ref[...] = gy * gx
            cp = pltpu.make_async_copy(
                heat_ref, out_ref.at[pl.ds(b, 1), pl.ds(0, 1)], sem)
            cp.start()
            cp.wait()
        return c

    lax.fori_loop(0, B, step, 0)


def kernel(images, gt, mask):
    mask_i = mask.astype(jnp.int32)
    return pl.pallas_call(
        _body,
        grid=(1,),
        in_specs=[
            pl.BlockSpec(memory_space=pltpu.SMEM),
            pl.BlockSpec(memory_space=pltpu.SMEM),
            pl.BlockSpec(memory_space=pl.ANY),
        ],
        out_specs=pl.BlockSpec(memory_space=pl.ANY),
        out_shape=jax.ShapeDtypeStruct((B, C, H, W), jnp.float32),
        input_output_aliases={2: 0},
        scratch_shapes=[
            pltpu.VMEM((1, 1, H, W), jnp.float32),
            pltpu.SemaphoreType.DMA,
        ],
    )(mask_i, gt, images)


# speedup vs baseline: 1.0474x; 1.0474x over previous
import jax
import jax.numpy as jnp
from jax import lax
from jax.experimental import pallas as pl
from jax.experimental.pallas import tpu as pltpu

SIGMA = 5.0
B, C, H, W = 128, 3, 384, 384
INV = 1.0 / (2.0 * SIGMA * SIGMA)
NSLOT = 4


def _body(mask_ref, gt_ref, img_ref, out_ref, heat_ref, *sems):
    def start_slot(s, b):
        for j in range(NSLOT):
            @pl.when(s == j)
            def _():
                pltpu.make_async_copy(
                    heat_ref.at[pl.ds(j, 1)],
                    out_ref.at[pl.ds(b, 1), pl.ds(0, 1)], sems[j]).start()

    def wait_slot(s):
        for j in range(NSLOT):
            @pl.when(s == j)
            def _():
                pltpu.make_async_copy(
                    heat_ref.at[pl.ds(j, 1)],
                    out_ref.at[pl.ds(0, 1), pl.ds(0, 1)], sems[j]).wait()

    def loop_body(b, n):
        def masked(n):
            s = lax.rem(n, NSLOT)

            @pl.when(n >= NSLOT)
            def _():
                wait_slot(s)

            x0 = gt_ref[b, 0]
            y0 = gt_ref[b, 1]
            xs = lax.broadcasted_iota(jnp.int32, (1, 1, 1, W), 3).astype(jnp.float32)
            ys = lax.broadcasted_iota(jnp.int32, (1, 1, H, 1), 2).astype(jnp.float32)
            gx = jnp.exp(-((xs - x0) ** 2) * INV)
            gy = jnp.exp(-((ys - y0) ** 2) * INV)
            heat_ref[pl.ds(s, 1)] = gy * gx
            start_slot(s, b)
            return n + 1

        return lax.cond(mask_ref[b] != 0, masked, lambda n: n, n)

    n = lax.fori_loop(0, B, loop_body, jnp.int32(0))

    def drain(k, carry):
        @pl.when(k < lax.min(n, jnp.int32(NSLOT)))
        def _():
            first = lax.max(jnp.int32(0), n - NSLOT)
            wait_slot(lax.rem(first + k, NSLOT))
        return carry

    lax.fori_loop(0, NSLOT, drain, 0)


def kernel(images, gt, mask):
    mask_i = mask.astype(jnp.int32)
    return pl.pallas_call(
        _body,
        grid=(1,),
        in_specs=[
            pl.BlockSpec(memory_space=pltpu.SMEM),
            pl.BlockSpec(memory_space=pltpu.SMEM),
            pl.BlockSpec(memory_space=pl.ANY),
        ],
        out_specs=pl.BlockSpec(memory_space=pl.ANY),
        out_shape=jax.ShapeDtypeStruct((B, C, H, W), jnp.float32),
        input_output_aliases={2: 0},
        scratch_shapes=[
            pltpu.VMEM((NSLOT, 1, H, W), jnp.float32),
        ] + [pltpu.SemaphoreType.DMA] * NSLOT,
    )(mask_i, gt, images)
